# Initial kernel scaffold; baseline (speedup 1.0000x reference)
import jax
import jax.numpy as jnp
from jax.experimental import pallas as pl


def kernel(feat, proposals, W_cls, b_cls, W_reg, b_reg):
    raise NotImplementedError("write your pallas kernel here")


# speedup vs baseline: 1.5941x; 1.5941x over previous
import functools
import math

import jax
import jax.lax as lax
import jax.numpy as jnp
from jax.experimental import pallas as pl
from jax.experimental.pallas import tpu as pltpu

_N = 5000
_C = 256
_H = 200
_W = 304
_NC = 80
_SCORE_THR = 0.05
_IOU_THR = 0.5
_MAX_PER_IMG = 100
_PRE_K = 1000
_STRIDE = 4.0
_IMG_W = _W * _STRIDE
_IMG_H = _H * _STRIDE
_MAX_RATIO = abs(math.log(16.0 / 1000.0))
_P = 1024
_GB = 8
_HB = 1000


def _gather_kernel(idx_ref, *refs):
    del idx_ref
    ins = refs[:_GB]
    out = refs[_GB]
    for k in range(_GB):
        out[k : k + 1, :] = ins[k][0]


def _gather_rows(lin_idx, featmap):
    in_specs = [
        pl.BlockSpec(
            (1, 1, _C), functools.partial(lambda k, i, idx: (idx[_GB * i + k], 0, 0), k)
        )
        for k in range(_GB)
    ]
    return pl.pallas_call(
        _gather_kernel,
        grid_spec=pltpu.PrefetchScalarGridSpec(
            num_scalar_prefetch=1,
            grid=(_N // _GB,),
            in_specs=in_specs,
            out_specs=pl.BlockSpec((_GB, _C), lambda i, idx: (i, 0)),
        ),
        out_shape=jax.ShapeDtypeStruct((_N, _C), jnp.float32),
    )(lin_idx, *([featmap.reshape(_H * _W, 1, _C)] * _GB))


def _head_kernel(rows_ref, prop_ref, wc_ref, bc_ref, wr_ref, br_ref, sc_out, box_out):
    rf = rows_ref[...]
    logits = jnp.dot(rf, wc_ref[...], preferred_element_type=jnp.float32) + bc_ref[...]
    m = jnp.max(logits, axis=1, keepdims=True)
    e = jnp.exp(logits - m)
    sm = e / jnp.sum(e, axis=1, keepdims=True)
    sc_out[...] = jnp.where(sm > _SCORE_THR, sm, 0.0)

    reg = jnp.dot(rf, wr_ref[...], preferred_element_type=jnp.float32) + br_ref[...]
    p = prop_ref[...]
    px = (p[:, 0:1] + p[:, 2:3]) * 0.5
    py = (p[:, 1:2] + p[:, 3:4]) * 0.5
    pw = p[:, 2:3] - p[:, 0:1]
    ph = p[:, 3:4] - p[:, 1:2]
    dx = reg[:, 0:128] * 0.1
    dy = reg[:, 128:256] * 0.1
    dw = jnp.clip(reg[:, 256:384] * 0.2, -_MAX_RATIO, _MAX_RATIO)
    dh = jnp.clip(reg[:, 384:512] * 0.2, -_MAX_RATIO, _MAX_RATIO)
    gx = px + pw * dx
    gy = py + ph * dy
    gw = pw * jnp.exp(dw)
    gh = ph * jnp.exp(dh)
    box_out[:, 0:128] = jnp.clip(gx - gw * 0.5, 0.0, _IMG_W)
    box_out[:, 128:256] = jnp.clip(gy - gh * 0.5, 0.0, _IMG_H)
    box_out[:, 256:384] = jnp.clip(gx + gw * 0.5, 0.0, _IMG_W)
    box_out[:, 384:512] = jnp.clip(gy + gh * 0.5, 0.0, _IMG_H)


def _head(rows, proposals, wc, bc, wr, br):
    grid = (_N // _HB,)
    return pl.pallas_call(
        _head_kernel,
        grid=grid,
        in_specs=[
            pl.BlockSpec((_HB, _C), lambda i: (i, 0)),
            pl.BlockSpec((_HB, 4), lambda i: (i, 0)),
            pl.BlockSpec((_C, 128), lambda i: (0, 0)),
            pl.BlockSpec((1, 128), lambda i: (0, 0)),
            pl.BlockSpec((_C, 512), lambda i: (0, 0)),
            pl.BlockSpec((1, 512), lambda i: (0, 0)),
        ],
        out_specs=[
            pl.BlockSpec((_HB, 128), lambda i: (i, 0)),
            pl.BlockSpec((_HB, 512), lambda i: (i, 0)),
        ],
        out_shape=[
            jax.ShapeDtypeStruct((_N, 128), jnp.float32),
            jax.ShapeDtypeStruct((_N, 512), jnp.float32),
        ],
    )(rows, proposals, wc, bc, wr, br)


def _nms_kernel(off_ref, offT_ref, score_ref, keep_out, s_ref):
    x1r = offT_ref[0:1, :]
    y1r = offT_ref[1:2, :]
    x2r = offT_ref[2:3, :]
    y2r = offT_ref[3:4, :]
    a_row = jnp.maximum(x2r - x1r, 0.0) * jnp.maximum(y2r - y1r, 0.0)
    col_ids = lax.broadcasted_iota(jnp.int32, (1, _P), 1)

    blk = 128
    for b in range(_P // blk):
        x1c = off_ref[b * blk : (b + 1) * blk, 0:1]
        y1c = off_ref[b * blk : (b + 1) * blk, 1:2]
        x2c = off_ref[b * blk : (b + 1) * blk, 2:3]
        y2c = off_ref[b * blk : (b + 1) * blk, 3:4]
        a_col = jnp.maximum(x2c - x1c, 0.0) * jnp.maximum(y2c - y1c, 0.0)
        xx1 = jnp.maximum(x1c, x1r)
        yy1 = jnp.maximum(y1c, y1r)
        xx2 = jnp.minimum(x2c, x2r)
        yy2 = jnp.minimum(y2c, y2r)
        inter = jnp.maximum(xx2 - xx1, 0.0) * jnp.maximum(yy2 - yy1, 0.0)
        iou = inter / jnp.maximum(a_col + a_row - inter, 1e-6)
        row_ids = b * blk + lax.broadcasted_iota(jnp.int32, (blk, 1), 0)
        s_ref[b * blk : (b + 1) * blk, :] = jnp.where(
            (iou > _IOU_THR) & (col_ids > row_ids), 1.0, 0.0
        )

    keep0 = jnp.where(score_ref[...] > _SCORE_THR, 1.0, 0.0)

    def body(i, keep):
        sup = s_ref[pl.ds(i, 1), :]
        ki = jnp.sum(jnp.where(col_ids == i, keep, 0.0), axis=1, keepdims=True)
        return keep * (1.0 - sup * ki)

    keep_out[...] = lax.fori_loop(0, _P, body, keep0)


def _nms(off, offT, scores):
    return pl.pallas_call(
        _nms_kernel,
        in_specs=[
            pl.BlockSpec((_P, 4), lambda: (0, 0)),
            pl.BlockSpec((4, _P), lambda: (0, 0)),
            pl.BlockSpec((1, _P), lambda: (0, 0)),
        ],
        out_specs=pl.BlockSpec((1, _P), lambda: (0, 0)),
        out_shape=jax.ShapeDtypeStruct((1, _P), jnp.float32),
        scratch_shapes=[pltpu.VMEM((_P, _P), jnp.float32)],
    )(off, offT, scores)


def kernel(feat, proposals, W_cls, b_cls, W_reg, b_reg):
    featmap = jnp.transpose(feat[0], (1, 2, 0)).reshape(_H * _W, _C)
    cx = (proposals[:, 0] + proposals[:, 2]) * 0.5 / _STRIDE
    cy = (proposals[:, 1] + proposals[:, 3]) * 0.5 / _STRIDE
    ix = jnp.clip(jnp.round(cx), 0, _W - 1).astype(jnp.int32)
    iy = jnp.clip(jnp.round(cy), 0, _H - 1).astype(jnp.int32)
    lin = iy * _W + ix

    rows = _gather_rows(lin, featmap)

    wc = jnp.zeros((_C, 128), jnp.float32).at[:, : _NC + 1].set(W_cls)
    bc = jnp.full((1, 128), -1e30, jnp.float32).at[0, : _NC + 1].set(b_cls)
    wr_g = jnp.transpose(W_reg.reshape(_C, _NC, 4), (0, 2, 1))
    wr = jnp.zeros((_C, 4, 128), jnp.float32).at[:, :, :_NC].set(wr_g).reshape(_C, 512)
    br_g = jnp.transpose(b_reg.reshape(_NC, 4), (1, 0))
    br = jnp.zeros((4, 128), jnp.float32).at[:, :_NC].set(br_g).reshape(1, 512)

    sc, bx = _head(rows, proposals, wc, bc, wr, br)

    masked = sc[:, :_NC]
    flat = masked.reshape(-1)
    top_scores, top_idx = lax.top_k(flat, _PRE_K)
    cls_idx = (top_idx % _NC).astype(jnp.int32)
    boxes = jnp.transpose(bx.reshape(_N, 4, 128)[:, :, :_NC], (0, 2, 1)).reshape(-1, 4)
    cand = boxes[top_idx]
    off = cand + (cls_idx.astype(cand.dtype) * (max(_IMG_W, _IMG_H) + 1.0))[:, None]

    offp = jnp.zeros((_P, 4), jnp.float32).at[:_PRE_K].set(off)
    offT = offp.T
    sp = jnp.full((1, _P), -1.0, jnp.float32).at[0, :_PRE_K].set(top_scores)

    keep = _nms(offp, offT, sp)
    keepb = keep[0, :_PRE_K] > 0.5

    sel_scores, sel = lax.top_k(jnp.where(keepb, top_scores, -1.0), _MAX_PER_IMG)
    det_boxes = cand[sel]
    det_scores = jnp.maximum(sel_scores, 0.0)
    det_classes = cls_idx[sel]
    num_det = jnp.sum(sel_scores > _SCORE_THR).astype(jnp.int32)
    return num_det, det_boxes[None], det_scores[None], det_classes[None]


# speedup vs baseline: 1.8070x; 1.1335x over previous
import functools
import math

import jax
import jax.lax as lax
import jax.numpy as jnp
from jax.experimental import pallas as pl
from jax.experimental.pallas import tpu as pltpu

_N = 5000
_C = 256
_H = 200
_W = 304
_NC = 80
_SCORE_THR = 0.05
_IOU_THR = 0.5
_MAX_PER_IMG = 100
_PRE_K = 1000
_STRIDE = 4.0
_IMG_W = _W * _STRIDE
_IMG_H = _H * _STRIDE
_MAX_RATIO = abs(math.log(16.0 / 1000.0))
_P = 1024
_GB = 16
_NPAD = 5008
_HB = 1000


def _gather_kernel(idx_ref, *refs):
    del idx_ref
    ins = refs[:_GB]
    out = refs[_GB]
    for k in range(_GB):
        out[k : k + 1, :] = ins[k][0]


def _gather_rows(lin_idx, featmap):
    in_specs = [
        pl.BlockSpec(
            (1, 1, _C), functools.partial(lambda k, i, idx: (idx[_GB * i + k], 0, 0), k)
        )
        for k in range(_GB)
    ]
    return pl.pallas_call(
        _gather_kernel,
        grid_spec=pltpu.PrefetchScalarGridSpec(
            num_scalar_prefetch=1,
            grid=(_NPAD // _GB,),
            in_specs=in_specs,
            out_specs=pl.BlockSpec((_GB, _C), lambda i, idx: (i, 0)),
        ),
        out_shape=jax.ShapeDtypeStruct((_NPAD, _C), jnp.float32),
    )(lin_idx, *([featmap.reshape(_H * _W, 1, _C)] * _GB))


def _head_kernel(rows_ref, prop_ref, wc_ref, bc_ref, wr_ref, br_ref, sc_out, box_out):
    rf = rows_ref[...]
    logits = jnp.dot(rf, wc_ref[...], preferred_element_type=jnp.float32) + bc_ref[...]
    m = jnp.max(logits, axis=1, keepdims=True)
    e = jnp.exp(logits - m)
    sm = e / jnp.sum(e, axis=1, keepdims=True)
    sc_out[...] = jnp.where(sm > _SCORE_THR, sm, 0.0)

    reg = jnp.dot(rf, wr_ref[...], preferred_element_type=jnp.float32) + br_ref[...]
    p = prop_ref[...]
    px = (p[:, 0:1] + p[:, 2:3]) * 0.5
    py = (p[:, 1:2] + p[:, 3:4]) * 0.5
    pw = p[:, 2:3] - p[:, 0:1]
    ph = p[:, 3:4] - p[:, 1:2]
    dx = reg[:, 0:128] * 0.1
    dy = reg[:, 128:256] * 0.1
    dw = jnp.clip(reg[:, 256:384] * 0.2, -_MAX_RATIO, _MAX_RATIO)
    dh = jnp.clip(reg[:, 384:512] * 0.2, -_MAX_RATIO, _MAX_RATIO)
    gx = px + pw * dx
    gy = py + ph * dy
    gw = pw * jnp.exp(dw)
    gh = ph * jnp.exp(dh)
    box_out[:, 0:128] = jnp.clip(gx - gw * 0.5, 0.0, _IMG_W)
    box_out[:, 128:256] = jnp.clip(gy - gh * 0.5, 0.0, _IMG_H)
    box_out[:, 256:384] = jnp.clip(gx + gw * 0.5, 0.0, _IMG_W)
    box_out[:, 384:512] = jnp.clip(gy + gh * 0.5, 0.0, _IMG_H)


def _head(rows, proposals, wc, bc, wr, br):
    grid = (_N // _HB,)
    return pl.pallas_call(
        _head_kernel,
        grid=grid,
        in_specs=[
            pl.BlockSpec((_HB, _C), lambda i: (i, 0)),
            pl.BlockSpec((_HB, 4), lambda i: (i, 0)),
            pl.BlockSpec((_C, 128), lambda i: (0, 0)),
            pl.BlockSpec((1, 128), lambda i: (0, 0)),
            pl.BlockSpec((_C, 512), lambda i: (0, 0)),
            pl.BlockSpec((1, 512), lambda i: (0, 0)),
        ],
        out_specs=[
            pl.BlockSpec((_HB, 128), lambda i: (i, 0)),
            pl.BlockSpec((_HB, 512), lambda i: (i, 0)),
        ],
        out_shape=[
            jax.ShapeDtypeStruct((_N, 128), jnp.float32),
            jax.ShapeDtypeStruct((_N, 512), jnp.float32),
        ],
    )(rows, proposals, wc, bc, wr, br)


def _nms_kernel(off_ref, offT_ref, score_ref, keep_out, s_ref):
    x1r = offT_ref[0:1, :]
    y1r = offT_ref[1:2, :]
    x2r = offT_ref[2:3, :]
    y2r = offT_ref[3:4, :]
    a_row = jnp.maximum(x2r - x1r, 0.0) * jnp.maximum(y2r - y1r, 0.0)
    col_ids = lax.broadcasted_iota(jnp.int32, (1, _P), 1)

    blk = 128
    for b in range(_P // blk):
        x1c = off_ref[b * blk : (b + 1) * blk, 0:1]
        y1c = off_ref[b * blk : (b + 1) * blk, 1:2]
        x2c = off_ref[b * blk : (b + 1) * blk, 2:3]
        y2c = off_ref[b * blk : (b + 1) * blk, 3:4]
        a_col = jnp.maximum(x2c - x1c, 0.0) * jnp.maximum(y2c - y1c, 0.0)
        xx1 = jnp.maximum(x1c, x1r)
        yy1 = jnp.maximum(y1c, y1r)
        xx2 = jnp.minimum(x2c, x2r)
        yy2 = jnp.minimum(y2c, y2r)
        inter = jnp.maximum(xx2 - xx1, 0.0) * jnp.maximum(yy2 - yy1, 0.0)
        iou = inter / jnp.maximum(a_col + a_row - inter, 1e-6)
        row_ids = b * blk + lax.broadcasted_iota(jnp.int32, (blk, 1), 0)
        s_ref[b * blk : (b + 1) * blk, :] = jnp.where(
            (iou > _IOU_THR) & (col_ids > row_ids), 1.0, 0.0
        )

    keep0 = jnp.where(score_ref[...] > _SCORE_THR, 1.0, 0.0)

    def body(i, keep):
        sup = s_ref[pl.ds(i, 1), :]
        ki = jnp.sum(jnp.where(col_ids == i, keep, 0.0), axis=1, keepdims=True)
        return keep * (1.0 - sup * ki)

    keep_out[...] = lax.fori_loop(0, _P, body, keep0)


def _nms(off, offT, scores):
    return pl.pallas_call(
        _nms_kernel,
        in_specs=[
            pl.BlockSpec((_P, 4), lambda: (0, 0)),
            pl.BlockSpec((4, _P), lambda: (0, 0)),
            pl.BlockSpec((1, _P), lambda: (0, 0)),
        ],
        out_specs=pl.BlockSpec((1, _P), lambda: (0, 0)),
        out_shape=jax.ShapeDtypeStruct((1, _P), jnp.float32),
        scratch_shapes=[pltpu.VMEM((_P, _P), jnp.float32)],
    )(off, offT, scores)


def kernel(feat, proposals, W_cls, b_cls, W_reg, b_reg):
    featmap = jnp.transpose(feat[0], (1, 2, 0)).reshape(_H * _W, _C)
    cx = (proposals[:, 0] + proposals[:, 2]) * 0.5 / _STRIDE
    cy = (proposals[:, 1] + proposals[:, 3]) * 0.5 / _STRIDE
    ix = jnp.clip(jnp.round(cx), 0, _W - 1).astype(jnp.int32)
    iy = jnp.clip(jnp.round(cy), 0, _H - 1).astype(jnp.int32)
    lin = iy * _W + ix
    lin_p = jnp.zeros((_NPAD,), jnp.int32).at[:_N].set(lin)

    rows = _gather_rows(lin_p, featmap)[:_N]

    wc = jnp.zeros((_C, 128), jnp.float32).at[:, : _NC + 1].set(W_cls)
    bc = jnp.full((1, 128), -1e30, jnp.float32).at[0, : _NC + 1].set(b_cls)
    wr_g = jnp.transpose(W_reg.reshape(_C, _NC, 4), (0, 2, 1))
    wr = jnp.zeros((_C, 4, 128), jnp.float32).at[:, :, :_NC].set(wr_g).reshape(_C, 512)
    br_g = jnp.transpose(b_reg.reshape(_NC, 4), (1, 0))
    br = jnp.zeros((4, 128), jnp.float32).at[:, :_NC].set(br_g).reshape(1, 512)

    sc, bx = _head(rows, proposals, wc, bc, wr, br)

    masked = sc[:, :_NC]
    flat = masked.reshape(-1)
    top_scores, top_idx = lax.top_k(flat, _PRE_K)
    cls_idx = (top_idx % _NC).astype(jnp.int32)
    boxes = jnp.transpose(bx.reshape(_N, 4, 128)[:, :, :_NC], (0, 2, 1)).reshape(-1, 4)
    cand = boxes[top_idx]
    off = cand + (cls_idx.astype(cand.dtype) * (max(_IMG_W, _IMG_H) + 1.0))[:, None]

    offp = jnp.zeros((_P, 4), jnp.float32).at[:_PRE_K].set(off)
    offT = offp.T
    sp = jnp.full((1, _P), -1.0, jnp.float32).at[0, :_PRE_K].set(top_scores)

    keep = _nms(offp, offT, sp)
    keepb = keep[0, :_PRE_K] > 0.5

    sel_scores, sel = lax.top_k(jnp.where(keepb, top_scores, -1.0), _MAX_PER_IMG)
    det_boxes = cand[sel]
    det_scores = jnp.maximum(sel_scores, 0.0)
    det_classes = cls_idx[sel]
    num_det = jnp.sum(sel_scores > _SCORE_THR).astype(jnp.int32)
    return num_det, det_boxes[None], det_scores[None], det_classes[None]


# speedup vs baseline: 1.8727x; 1.0364x over previous
import functools
import math

import jax
import jax.lax as lax
import jax.numpy as jnp
from jax.experimental import pallas as pl
from jax.experimental.pallas import tpu as pltpu

_N = 5000
_C = 256
_H = 200
_W = 304
_NC = 80
_SCORE_THR = 0.05
_IOU_THR = 0.5
_MAX_PER_IMG = 100
_PRE_K = 1000
_STRIDE = 4.0
_IMG_W = _W * _STRIDE
_IMG_H = _H * _STRIDE
_MAX_RATIO = abs(math.log(16.0 / 1000.0))
_P = 1024
_GB = 32
_NPAD = 5024
_HB = 1000


def _gather_kernel(idx_ref, *refs):
    del idx_ref
    ins = refs[:_GB]
    out = refs[_GB]
    for k in range(_GB):
        out[k : k + 1, :] = ins[k][0]


def _gather_rows(lin_idx, featmap):
    in_specs = [
        pl.BlockSpec(
            (1, 1, _C), functools.partial(lambda k, i, idx: (idx[_GB * i + k], 0, 0), k)
        )
        for k in range(_GB)
    ]
    return pl.pallas_call(
        _gather_kernel,
        grid_spec=pltpu.PrefetchScalarGridSpec(
            num_scalar_prefetch=1,
            grid=(_NPAD // _GB,),
            in_specs=in_specs,
            out_specs=pl.BlockSpec((_GB, _C), lambda i, idx: (i, 0)),
        ),
        out_shape=jax.ShapeDtypeStruct((_NPAD, _C), jnp.float32),
    )(lin_idx, *([featmap.reshape(_H * _W, 1, _C)] * _GB))


def _head_kernel(rows_ref, prop_ref, wc_ref, bc_ref, wr_ref, br_ref, sc_out, box_out):
    rf = rows_ref[...]
    logits = jnp.dot(rf, wc_ref[...], preferred_element_type=jnp.float32) + bc_ref[...]
    m = jnp.max(logits, axis=1, keepdims=True)
    e = jnp.exp(logits - m)
    sm = e / jnp.sum(e, axis=1, keepdims=True)
    sc_out[...] = jnp.where(sm > _SCORE_THR, sm, 0.0)

    reg = jnp.dot(rf, wr_ref[...], preferred_element_type=jnp.float32) + br_ref[...]
    p = prop_ref[...]
    px = (p[:, 0:1] + p[:, 2:3]) * 0.5
    py = (p[:, 1:2] + p[:, 3:4]) * 0.5
    pw = p[:, 2:3] - p[:, 0:1]
    ph = p[:, 3:4] - p[:, 1:2]
    dx = reg[:, 0:128] * 0.1
    dy = reg[:, 128:256] * 0.1
    dw = jnp.clip(reg[:, 256:384] * 0.2, -_MAX_RATIO, _MAX_RATIO)
    dh = jnp.clip(reg[:, 384:512] * 0.2, -_MAX_RATIO, _MAX_RATIO)
    gx = px + pw * dx
    gy = py + ph * dy
    gw = pw * jnp.exp(dw)
    gh = ph * jnp.exp(dh)
    box_out[:, 0:128] = jnp.clip(gx - gw * 0.5, 0.0, _IMG_W)
    box_out[:, 128:256] = jnp.clip(gy - gh * 0.5, 0.0, _IMG_H)
    box_out[:, 256:384] = jnp.clip(gx + gw * 0.5, 0.0, _IMG_W)
    box_out[:, 384:512] = jnp.clip(gy + gh * 0.5, 0.0, _IMG_H)


def _head(rows, proposals, wc, bc, wr, br):
    grid = (_N // _HB,)
    return pl.pallas_call(
        _head_kernel,
        grid=grid,
        in_specs=[
            pl.BlockSpec((_HB, _C), lambda i: (i, 0)),
            pl.BlockSpec((_HB, 4), lambda i: (i, 0)),
            pl.BlockSpec((_C, 128), lambda i: (0, 0)),
            pl.BlockSpec((1, 128), lambda i: (0, 0)),
            pl.BlockSpec((_C, 512), lambda i: (0, 0)),
            pl.BlockSpec((1, 512), lambda i: (0, 0)),
        ],
        out_specs=[
            pl.BlockSpec((_HB, 128), lambda i: (i, 0)),
            pl.BlockSpec((_HB, 512), lambda i: (i, 0)),
        ],
        out_shape=[
            jax.ShapeDtypeStruct((_N, 128), jnp.float32),
            jax.ShapeDtypeStruct((_N, 512), jnp.float32),
        ],
    )(rows, proposals, wc, bc, wr, br)


def _nms_kernel(off_ref, offT_ref, score_ref, keep_out, s_ref):
    x1r = offT_ref[0:1, :]
    y1r = offT_ref[1:2, :]
    x2r = offT_ref[2:3, :]
    y2r = offT_ref[3:4, :]
    a_row = jnp.maximum(x2r - x1r, 0.0) * jnp.maximum(y2r - y1r, 0.0)
    col_ids = lax.broadcasted_iota(jnp.int32, (1, _P), 1)

    blk = 128
    for b in range(_P // blk):
        x1c = off_ref[b * blk : (b + 1) * blk, 0:1]
        y1c = off_ref[b * blk : (b + 1) * blk, 1:2]
        x2c = off_ref[b * blk : (b + 1) * blk, 2:3]
        y2c = off_ref[b * blk : (b + 1) * blk, 3:4]
        a_col = jnp.maximum(x2c - x1c, 0.0) * jnp.maximum(y2c - y1c, 0.0)
        xx1 = jnp.maximum(x1c, x1r)
        yy1 = jnp.maximum(y1c, y1r)
        xx2 = jnp.minimum(x2c, x2r)
        yy2 = jnp.minimum(y2c, y2r)
        inter = jnp.maximum(xx2 - xx1, 0.0) * jnp.maximum(yy2 - yy1, 0.0)
        iou = inter / jnp.maximum(a_col + a_row - inter, 1e-6)
        row_ids = b * blk + lax.broadcasted_iota(jnp.int32, (blk, 1), 0)
        s_ref[b * blk : (b + 1) * blk, :] = jnp.where(
            (iou > _IOU_THR) & (col_ids > row_ids), 1.0, 0.0
        )

    keep0 = jnp.where(score_ref[...] > _SCORE_THR, 1.0, 0.0)

    def body(i, keep):
        sup = s_ref[pl.ds(i, 1), :]
        ki = jnp.sum(jnp.where(col_ids == i, keep, 0.0), axis=1, keepdims=True)
        return keep * (1.0 - sup * ki)

    keep_out[...] = lax.fori_loop(0, _P, body, keep0)


def _nms(off, offT, scores):
    return pl.pallas_call(
        _nms_kernel,
        in_specs=[
            pl.BlockSpec((_P, 4), lambda: (0, 0)),
            pl.BlockSpec((4, _P), lambda: (0, 0)),
            pl.BlockSpec((1, _P), lambda: (0, 0)),
        ],
        out_specs=pl.BlockSpec((1, _P), lambda: (0, 0)),
        out_shape=jax.ShapeDtypeStruct((1, _P), jnp.float32),
        scratch_shapes=[pltpu.VMEM((_P, _P), jnp.float32)],
    )(off, offT, scores)


def kernel(feat, proposals, W_cls, b_cls, W_reg, b_reg):
    featmap = jnp.transpose(feat[0], (1, 2, 0)).reshape(_H * _W, _C)
    cx = (proposals[:, 0] + proposals[:, 2]) * 0.5 / _STRIDE
    cy = (proposals[:, 1] + proposals[:, 3]) * 0.5 / _STRIDE
    ix = jnp.clip(jnp.round(cx), 0, _W - 1).astype(jnp.int32)
    iy = jnp.clip(jnp.round(cy), 0, _H - 1).astype(jnp.int32)
    lin = iy * _W + ix
    lin_p = jnp.zeros((_NPAD,), jnp.int32).at[:_N].set(lin)

    rows = _gather_rows(lin_p, featmap)[:_N]

    wc = jnp.zeros((_C, 128), jnp.float32).at[:, : _NC + 1].set(W_cls)
    bc = jnp.full((1, 128), -1e30, jnp.float32).at[0, : _NC + 1].set(b_cls)
    wr_g = jnp.transpose(W_reg.reshape(_C, _NC, 4), (0, 2, 1))
    wr = jnp.zeros((_C, 4, 128), jnp.float32).at[:, :, :_NC].set(wr_g).reshape(_C, 512)
    br_g = jnp.transpose(b_reg.reshape(_NC, 4), (1, 0))
    br = jnp.zeros((4, 128), jnp.float32).at[:, :_NC].set(br_g).reshape(1, 512)

    sc, bx = _head(rows, proposals, wc, bc, wr, br)

    masked = sc[:, :_NC]
    flat = masked.reshape(-1)
    top_scores, top_idx = lax.top_k(flat, _PRE_K)
    cls_idx = (top_idx % _NC).astype(jnp.int32)
    boxes = jnp.transpose(bx.reshape(_N, 4, 128)[:, :, :_NC], (0, 2, 1)).reshape(-1, 4)
    cand = boxes[top_idx]
    off = cand + (cls_idx.astype(cand.dtype) * (max(_IMG_W, _IMG_H) + 1.0))[:, None]

    offp = jnp.zeros((_P, 4), jnp.float32).at[:_PRE_K].set(off)
    offT = offp.T
    sp = jnp.full((1, _P), -1.0, jnp.float32).at[0, :_PRE_K].set(top_scores)

    keep = _nms(offp, offT, sp)
    keepb = keep[0, :_PRE_K] > 0.5

    sel_scores, sel = lax.top_k(jnp.where(keepb, top_scores, -1.0), _MAX_PER_IMG)
    det_boxes = cand[sel]
    det_scores = jnp.maximum(sel_scores, 0.0)
    det_classes = cls_idx[sel]
    num_det = jnp.sum(sel_scores > _SCORE_THR).astype(jnp.int32)
    return num_det, det_boxes[None], det_scores[None], det_classes[None]


# speedup vs baseline: 1.9020x; 1.0156x over previous
import functools
import math

import jax
import jax.lax as lax
import jax.numpy as jnp
from jax.experimental import pallas as pl
from jax.experimental.pallas import tpu as pltpu

_N = 5000
_C = 256
_H = 200
_W = 304
_NC = 80
_SCORE_THR = 0.05
_IOU_THR = 0.5
_MAX_PER_IMG = 100
_PRE_K = 1000
_STRIDE = 4.0
_IMG_W = _W * _STRIDE
_IMG_H = _H * _STRIDE
_MAX_RATIO = abs(math.log(16.0 / 1000.0))
_P = 1024
_GB = 32
_NPAD = 5024
_HB = 1000


def _gather_kernel(idx_ref, *refs):
    del idx_ref
    ins = refs[:_GB]
    out = refs[_GB]
    for k in range(_GB):
        out[k : k + 1, :] = ins[k][0]


def _gather_rows(lin_idx, featmap):
    in_specs = [
        pl.BlockSpec(
            (1, 1, _C), functools.partial(lambda k, i, idx: (idx[_GB * i + k], 0, 0), k)
        )
        for k in range(_GB)
    ]
    return pl.pallas_call(
        _gather_kernel,
        grid_spec=pltpu.PrefetchScalarGridSpec(
            num_scalar_prefetch=1,
            grid=(_NPAD // _GB,),
            in_specs=in_specs,
            out_specs=pl.BlockSpec((_GB, _C), lambda i, idx: (i, 0)),
        ),
        out_shape=jax.ShapeDtypeStruct((_NPAD, _C), jnp.float32),
    )(lin_idx, *([featmap.reshape(_H * _W, 1, _C)] * _GB))


def _head_kernel(rows_ref, prop_ref, wc_ref, bc_ref, wr_ref, br_ref, sc_out, box_out):
    rf = rows_ref[...]
    logits = jnp.dot(rf, wc_ref[...], preferred_element_type=jnp.float32) + bc_ref[...]
    m = jnp.max(logits, axis=1, keepdims=True)
    e = jnp.exp(logits - m)
    sm = e / jnp.sum(e, axis=1, keepdims=True)
    sc_out[...] = jnp.where(sm > _SCORE_THR, sm, 0.0)

    reg = jnp.dot(rf, wr_ref[...], preferred_element_type=jnp.float32) + br_ref[...]
    p = prop_ref[...]
    px = (p[:, 0:1] + p[:, 2:3]) * 0.5
    py = (p[:, 1:2] + p[:, 3:4]) * 0.5
    pw = p[:, 2:3] - p[:, 0:1]
    ph = p[:, 3:4] - p[:, 1:2]
    dx = reg[:, 0:128] * 0.1
    dy = reg[:, 128:256] * 0.1
    dw = jnp.clip(reg[:, 256:384] * 0.2, -_MAX_RATIO, _MAX_RATIO)
    dh = jnp.clip(reg[:, 384:512] * 0.2, -_MAX_RATIO, _MAX_RATIO)
    gx = px + pw * dx
    gy = py + ph * dy
    gw = pw * jnp.exp(dw)
    gh = ph * jnp.exp(dh)
    box_out[:, 0:128] = jnp.clip(gx - gw * 0.5, 0.0, _IMG_W)
    box_out[:, 128:256] = jnp.clip(gy - gh * 0.5, 0.0, _IMG_H)
    box_out[:, 256:384] = jnp.clip(gx + gw * 0.5, 0.0, _IMG_W)
    box_out[:, 384:512] = jnp.clip(gy + gh * 0.5, 0.0, _IMG_H)


def _head(rows, proposals, wc, bc, wr, br):
    grid = (_N // _HB,)
    return pl.pallas_call(
        _head_kernel,
        grid=grid,
        in_specs=[
            pl.BlockSpec((_HB, _C), lambda i: (i, 0)),
            pl.BlockSpec((_HB, 4), lambda i: (i, 0)),
            pl.BlockSpec((_C, 128), lambda i: (0, 0)),
            pl.BlockSpec((1, 128), lambda i: (0, 0)),
            pl.BlockSpec((_C, 512), lambda i: (0, 0)),
            pl.BlockSpec((1, 512), lambda i: (0, 0)),
        ],
        out_specs=[
            pl.BlockSpec((_HB, 128), lambda i: (i, 0)),
            pl.BlockSpec((_HB, 512), lambda i: (i, 0)),
        ],
        out_shape=[
            jax.ShapeDtypeStruct((_N, 128), jnp.float32),
            jax.ShapeDtypeStruct((_N, 512), jnp.float32),
        ],
    )(rows, proposals, wc, bc, wr, br)


def _nms_kernel(off_ref, offT_ref, score_ref, keep_out, s_ref, sdiag_ref):
    x1r = offT_ref[0:1, :]
    y1r = offT_ref[1:2, :]
    x2r = offT_ref[2:3, :]
    y2r = offT_ref[3:4, :]
    a_row = jnp.maximum(x2r - x1r, 0.0) * jnp.maximum(y2r - y1r, 0.0)
    col_ids = lax.broadcasted_iota(jnp.int32, (1, _P), 1)

    blk = 128
    for b in range(_P // blk):
        x1c = off_ref[b * blk : (b + 1) * blk, 0:1]
        y1c = off_ref[b * blk : (b + 1) * blk, 1:2]
        x2c = off_ref[b * blk : (b + 1) * blk, 2:3]
        y2c = off_ref[b * blk : (b + 1) * blk, 3:4]
        a_col = jnp.maximum(x2c - x1c, 0.0) * jnp.maximum(y2c - y1c, 0.0)
        xx1 = jnp.maximum(x1c, x1r)
        yy1 = jnp.maximum(y1c, y1r)
        xx2 = jnp.minimum(x2c, x2r)
        yy2 = jnp.minimum(y2c, y2r)
        inter = jnp.maximum(xx2 - xx1, 0.0) * jnp.maximum(yy2 - yy1, 0.0)
        iou = inter / jnp.maximum(a_col + a_row - inter, 1e-6)
        row_ids = b * blk + lax.broadcasted_iota(jnp.int32, (blk, 1), 0)
        s_blk = jnp.where((iou > _IOU_THR) & (col_ids > row_ids), 1.0, 0.0)
        s_ref[b * blk : (b + 1) * blk, :] = s_blk
        sdiag_ref[b * blk : (b + 1) * blk, :] = s_blk[:, b * blk : (b + 1) * blk]

    keep = jnp.where(score_ref[...] > _SCORE_THR, 1.0, 0.0)
    iota_t = lax.broadcasted_iota(jnp.int32, (1, blk), 1)

    for t in range(_P // blk):
        tile = keep[:, t * blk : (t + 1) * blk]

        def body(i, tk):
            sup = sdiag_ref[pl.ds(t * blk + i, 1), :]
            ki = jnp.sum(jnp.where(iota_t == i, tk, 0.0), axis=1, keepdims=True)
            return tk * (1.0 - sup * ki)

        tile = lax.fori_loop(0, blk, body, tile)
        parts = []
        if t > 0:
            parts.append(keep[:, : t * blk])
        parts.append(tile)
        if t < _P // blk - 1:
            parts.append(keep[:, (t + 1) * blk :])
        keep = jnp.concatenate(parts, axis=1)
        srows = s_ref[t * blk : (t + 1) * blk, :]
        supp = jnp.dot(tile, srows, preferred_element_type=jnp.float32)
        keep = keep * jnp.where(supp > 0.0, 0.0, 1.0)

    keep_out[...] = keep


def _nms(off, offT, scores):
    return pl.pallas_call(
        _nms_kernel,
        in_specs=[
            pl.BlockSpec((_P, 4), lambda: (0, 0)),
            pl.BlockSpec((4, _P), lambda: (0, 0)),
            pl.BlockSpec((1, _P), lambda: (0, 0)),
        ],
        out_specs=pl.BlockSpec((1, _P), lambda: (0, 0)),
        out_shape=jax.ShapeDtypeStruct((1, _P), jnp.float32),
        scratch_shapes=[
            pltpu.VMEM((_P, _P), jnp.float32),
            pltpu.VMEM((_P, 128), jnp.float32),
        ],
    )(off, offT, scores)


def kernel(feat, proposals, W_cls, b_cls, W_reg, b_reg):
    featmap = jnp.transpose(feat[0], (1, 2, 0)).reshape(_H * _W, _C)
    cx = (proposals[:, 0] + proposals[:, 2]) * 0.5 / _STRIDE
    cy = (proposals[:, 1] + proposals[:, 3]) * 0.5 / _STRIDE
    ix = jnp.clip(jnp.round(cx), 0, _W - 1).astype(jnp.int32)
    iy = jnp.clip(jnp.round(cy), 0, _H - 1).astype(jnp.int32)
    lin = iy * _W + ix
    lin_p = jnp.zeros((_NPAD,), jnp.int32).at[:_N].set(lin)

    rows = _gather_rows(lin_p, featmap)[:_N]

    wc = jnp.zeros((_C, 128), jnp.float32).at[:, : _NC + 1].set(W_cls)
    bc = jnp.full((1, 128), -1e30, jnp.float32).at[0, : _NC + 1].set(b_cls)
    wr_g = jnp.transpose(W_reg.reshape(_C, _NC, 4), (0, 2, 1))
    wr = jnp.zeros((_C, 4, 128), jnp.float32).at[:, :, :_NC].set(wr_g).reshape(_C, 512)
    br_g = jnp.transpose(b_reg.reshape(_NC, 4), (1, 0))
    br = jnp.zeros((4, 128), jnp.float32).at[:, :_NC].set(br_g).reshape(1, 512)

    sc, bx = _head(rows, proposals, wc, bc, wr, br)

    masked = sc[:, :_NC]
    flat = masked.reshape(-1)
    top_scores, top_idx = lax.top_k(flat, _PRE_K)
    cls_idx = (top_idx % _NC).astype(jnp.int32)
    boxes = jnp.transpose(bx.reshape(_N, 4, 128)[:, :, :_NC], (0, 2, 1)).reshape(-1, 4)
    cand = boxes[top_idx]
    off = cand + (cls_idx.astype(cand.dtype) * (max(_IMG_W, _IMG_H) + 1.0))[:, None]

    offp = jnp.zeros((_P, 4), jnp.float32).at[:_PRE_K].set(off)
    offT = offp.T
    sp = jnp.full((1, _P), -1.0, jnp.float32).at[0, :_PRE_K].set(top_scores)

    keep = _nms(offp, offT, sp)
    keepb = keep[0, :_PRE_K] > 0.5

    sel_scores, sel = lax.top_k(jnp.where(keepb, top_scores, -1.0), _MAX_PER_IMG)
    det_boxes = cand[sel]
    det_scores = jnp.maximum(sel_scores, 0.0)
    det_classes = cls_idx[sel]
    num_det = jnp.sum(sel_scores > _SCORE_THR).astype(jnp.int32)
    return num_det, det_boxes[None], det_scores[None], det_classes[None]
